# baseline (speedup 1.0000x reference)
import functools

import jax
import jax.numpy as jnp
from jax.experimental import pallas as pl
from jax.experimental.pallas import tpu as pltpu

_PAD = 128


def _conv_body(x_ref, w_ref, b_ref, o_ref, xz_ref, *, H, W, L, taps):
    c = pl.program_id(1)
    Cin, HWp = xz_ref.shape
    HW = H * W
    Cout = o_ref.shape[1]

    @pl.when(c == 0)
    def _fill():
        dt = xz_ref.dtype
        xz_ref[:, 0:_PAD] = jnp.zeros((Cin, _PAD), dt)
        xz_ref[:, _PAD + HW:HWp] = jnp.zeros((Cin, HWp - _PAD - HW), dt)
        xz_ref[:, _PAD:_PAD + HW] = x_ref[0]

    l0 = c * L

    wi = jax.lax.broadcasted_iota(jnp.int32, (1, L), 1) % W
    mask_l = (wi > 0).astype(jnp.float32)
    mask_r = (wi < W - 1).astype(jnp.float32)

    band = xz_ref[:, pl.ds(pl.multiple_of(l0, L), L + 2 * _PAD)]

    acc = jnp.broadcast_to(b_ref[...], (Cout, L))
    for k, (dy, dx) in enumerate(taps):
        s = _PAD + dy * W + dx
        tap = jax.lax.slice_in_dim(band, s, s + L, axis=1)
        if dx < 0:
            tap = tap * mask_l
        elif dx > 0:
            tap = tap * mask_r
        acc = acc + jnp.dot(w_ref[k], tap, preferred_element_type=jnp.float32)
    o_ref[0] = acc


def _pick_chunk_rows(H, W, budget=512):
    for rc in range(min(H, budget // W), 0, -1):
        if H % rc == 0 and rc * W <= budget:
            return rc
    return 1


def kernel(x, weight, bias):
    N, Cin, H, W = x.shape
    Cout, Cin_w, kh, kw = weight.shape
    assert Cin_w == Cin and kh == 3 and kw == 3
    assert _PAD >= W + 1

    HW = H * W
    rc = _pick_chunk_rows(H, W)
    L = rc * W

    xf = x.reshape(N, Cin, HW)
    wt = jnp.transpose(weight, (2, 3, 0, 1)).reshape(kh * kw, Cout, Cin)
    b2 = bias.astype(jnp.float32).reshape(Cout, 1)

    taps = [(dy, dx) for dy in (-1, 0, 1) for dx in (-1, 0, 1)]
    body = functools.partial(_conv_body, H=H, W=W, L=L, taps=taps)

    grid = (N, HW // L)
    out = pl.pallas_call(
        body,
        out_shape=jax.ShapeDtypeStruct((N, Cout, HW), jnp.float32),
        grid=grid,
        in_specs=[
            pl.BlockSpec((1, Cin, HW), lambda n, c: (n, 0, 0)),
            pl.BlockSpec((kh * kw, Cout, Cin), lambda n, c: (0, 0, 0)),
            pl.BlockSpec((Cout, 1), lambda n, c: (0, 0)),
        ],
        out_specs=pl.BlockSpec((1, Cout, L), lambda n, c: (n, 0, c)),
        scratch_shapes=[pltpu.VMEM((Cin, _PAD + HW + _PAD), x.dtype)],
        compiler_params=pltpu.CompilerParams(
            dimension_semantics=("parallel", "arbitrary"),
            vmem_limit_bytes=48 * 1024 * 1024),
    )(xf, wt, b2)

    return out.reshape(N, Cout, H, W)


# speedup vs baseline: 1.9622x; 1.9622x over previous
import functools

import jax
import jax.numpy as jnp
from jax.experimental import pallas as pl
from jax.experimental.pallas import tpu as pltpu


def _conv_body(x_ref, w_ref, b_ref, o_ref, xz_ref, *, H, W, rc):
    c = pl.program_id(1)
    Hp, Wp, Cin = xz_ref.shape
    Cout = o_ref.shape[2]
    L = rc * W

    @pl.when(c == 0)
    def _fill():
        dt = xz_ref.dtype
        xz_ref[:, 0:1, :] = jnp.zeros((Hp, 1, Cin), dt)
        xz_ref[:, Wp - 1:Wp, :] = jnp.zeros((Hp, 1, Cin), dt)
        xz_ref[0:1, :, :] = jnp.zeros((1, Wp, Cin), dt)
        xz_ref[Hp - 1:Hp, :, :] = jnp.zeros((1, Wp, Cin), dt)
        xz_ref[1:Hp - 1, 1:Wp - 1, :] = x_ref[0]

    row0 = c * rc
    acc = jnp.broadcast_to(b_ref[...], (L, Cout))
    for dy in range(3):
        for dx in range(3):
            tap = xz_ref[pl.ds(row0 + dy, rc), pl.ds(dx, W), :]
            acc = acc + jnp.dot(tap.reshape(L, Cin), w_ref[dy * 3 + dx],
                                preferred_element_type=jnp.float32)
    o_ref[0] = acc


def _pick_chunk_rows(H, W, budget=512):
    for rc in range(min(H, budget // W), 0, -1):
        if H % rc == 0:
            return rc
    return 1


def kernel(x, weight, bias):
    N, Cin, H, W = x.shape
    Cout, Cin_w, kh, kw = weight.shape
    assert Cin_w == Cin and kh == 3 and kw == 3

    HW = H * W
    rc = _pick_chunk_rows(H, W)
    L = rc * W

    xh = jnp.transpose(x, (0, 2, 3, 1))
    wt = jnp.transpose(weight, (2, 3, 1, 0)).reshape(kh * kw, Cin, Cout)
    b2 = bias.astype(jnp.float32).reshape(1, Cout)

    body = functools.partial(_conv_body, H=H, W=W, rc=rc)

    grid = (N, H // rc)
    out = pl.pallas_call(
        body,
        out_shape=jax.ShapeDtypeStruct((N, HW, Cout), jnp.float32),
        grid=grid,
        in_specs=[
            pl.BlockSpec((1, H, W, Cin), lambda n, c: (n, 0, 0, 0)),
            pl.BlockSpec((kh * kw, Cin, Cout), lambda n, c: (0, 0, 0)),
            pl.BlockSpec((1, Cout), lambda n, c: (0, 0)),
        ],
        out_specs=pl.BlockSpec((1, L, Cout), lambda n, c: (n, c, 0)),
        scratch_shapes=[pltpu.VMEM((H + 2, W + 2, Cin), x.dtype)],
        compiler_params=pltpu.CompilerParams(
            dimension_semantics=("parallel", "arbitrary"),
            vmem_limit_bytes=48 * 1024 * 1024),
    )(xh, wt, b2)

    return jnp.transpose(out.reshape(N, H, W, Cout), (0, 3, 1, 2))
